# baseline (speedup 1.0000x reference)
import functools

import jax
import jax.numpy as jnp
from jax import lax
from jax.experimental import pallas as pl
from jax.experimental.pallas import tpu as pltpu
from jax.experimental.pallas import tpu_sc as plsc

VOCAB = 1000000
MAXLEN = 200
EMBED = 64
BATCH = 1024
SEQ = 200

LANES = 16
NW = 32
SEQ_PER_W = BATCH // NW
SEQ_PAD = 208
CHUNK0 = 128
CHUNK1 = SEQ - CHUNK0
NBUF = 4
N_CHUNKS = 2 * SEQ_PER_W
N_ROUNDS = N_CHUNKS // NBUF


def _chunk_geom(c):
  return c // 2, (c % 2) * CHUNK0, CHUNK1 if c % 2 else CHUNK0


def _make_kernel():
  mesh = plsc.VectorSubcoreMesh(core_axis_name="c", subcore_axis_name="s")

  @functools.partial(
      pl.kernel,
      mesh=mesh,
      compiler_params=pltpu.CompilerParams(
          use_tc_tiling_on_sc=False, needs_layout_passes=False
      ),
      out_type=jax.ShapeDtypeStruct((BATCH, SEQ, EMBED), jnp.float32),
      scratch_types=[
          pltpu.VMEM((SEQ, SEQ_PER_W), jnp.int32),
          pltpu.VMEM((SEQ_PER_W, SEQ_PAD), jnp.int32),
          pltpu.VMEM((MAXLEN, EMBED), jnp.float32),
          [pltpu.VMEM((CHUNK1 if b % 2 else CHUNK0, EMBED), jnp.float32)
           for b in range(NBUF)],
          [pltpu.SemaphoreType.DMA for _ in range(NBUF)],
          [pltpu.SemaphoreType.DMA for _ in range(NBUF)],
      ],
  )
  def embed(xt_hbm, tok_hbm, pos_hbm, out_hbm,
            idxt_v, idx_v, pos_v, rows, gsem, ssem):
    wid = lax.axis_index("s") * 2 + lax.axis_index("c")
    seq_base = wid * SEQ_PER_W
    pltpu.sync_copy(xt_hbm.at[:, pl.ds(seq_base, SEQ_PER_W)], idxt_v)
    pltpu.sync_copy(pos_hbm, pos_v)

    lane = lax.iota(jnp.int32, LANES)
    starts = tuple(range(0, SEQ - LANES, LANES)) + (SEQ - LANES,)

    def trans_body(sq, carry):
      col = jnp.full((LANES,), 0, jnp.int32) + sq
      for s0 in starts:
        vals = plsc.load_gather(idxt_v, [lane + s0, col])
        idx_v[sq, pl.ds(s0, LANES)] = vals
      return carry

    lax.fori_loop(0, SEQ_PER_W, trans_body, 0)

    def gather_start(b, seq, s0, ln):
      pltpu.async_copy(
          tok_hbm.at[idx_v.at[seq, pl.ds(s0, ln)]], rows[b], gsem[b]
      )

    def gather_wait(b, ln):
      pltpu.make_async_copy(tok_hbm.at[pl.ds(0, ln)], rows[b], gsem[b]).wait()

    def store_start(b, seq, s0, ln):
      pltpu.async_copy(
          rows[b], out_hbm.at[seq_base + seq, pl.ds(s0, ln), :], ssem[b]
      )

    def store_wait(b, seq, s0, ln):
      pltpu.make_async_copy(
          rows[b], out_hbm.at[seq_base + seq, pl.ds(s0, ln), :], ssem[b]
      ).wait()

    def add_pos(b, s0, ln):
      def row_body(r, rcarry):
        for d in range(EMBED // LANES):
          sl = pl.ds(d * LANES, LANES)
          plsc.addupdate(rows[b].at[r, sl], pos_v[s0 + r, sl])
        return rcarry

      lax.fori_loop(0, ln, row_body, 0, unroll=4)

    for b in range(NBUF):
      seq, s0, ln = _chunk_geom(b)
      gather_start(b, seq, s0, ln)

    def round_body(i, carry):
      c0 = i * NBUF
      for b in range(NBUF):
        s0 = (b % 2) * CHUNK0
        ln = CHUNK1 if b % 2 else CHUNK0
        seq = (c0 + b) // 2
        gather_wait(b, ln)
        add_pos(b, s0, ln)
        store_start(b, seq, s0, ln)
      for b in range(NBUF):
        s0 = (b % 2) * CHUNK0
        ln = CHUNK1 if b % 2 else CHUNK0
        seq = (c0 + b) // 2

        @pl.when(c0 + b + NBUF < N_CHUNKS)
        def _():
          store_wait(b, seq, s0, ln)
          gather_start(b, (c0 + b + NBUF) // 2, s0, ln)

      return carry

    lax.fori_loop(0, N_ROUNDS, round_body, 0)
    for b in range(NBUF):
      seq, s0, ln = _chunk_geom(N_CHUNKS - NBUF + b)
      store_wait(b, seq, s0, ln)

  return embed


_embed = _make_kernel()


def kernel(x, token_table, pos_table):
  xt = jnp.swapaxes(x, 0, 1).astype(jnp.int32)
  return _embed(xt, token_table, pos_table)
